# baseline (speedup 1.0000x reference)
import functools

import jax
import jax.numpy as jnp
from jax import lax
from jax.experimental import pallas as pl
from jax.experimental.pallas import tpu as pltpu
from jax.experimental.pallas import tpu_sc as plsc

_DIM = 64
_LANES = 16
_NC = 2
_NS = 16
_NW = _NC * _NS
_CHUNK = 128
_VPR = _DIM // _LANES


@functools.lru_cache(maxsize=None)
def _build(n_rows):
    rows_per_w = n_rows // _NW
    n_chunks = rows_per_w // _CHUNK
    mesh = plsc.VectorSubcoreMesh(core_axis_name="c", subcore_axis_name="s",
                                  num_cores=_NC, num_subcores=_NS)

    @functools.partial(
        pl.kernel,
        out_type=jax.ShapeDtypeStruct((n_rows, _DIM), jnp.float32),
        mesh=mesh,
        scratch_types=[
            pltpu.VMEM((_CHUNK,), jnp.int32),
            pltpu.VMEM((_CHUNK, _DIM), jnp.float32),
            pltpu.VMEM((_CHUNK, _DIM), jnp.float32),
            pltpu.SemaphoreType.DMA,
        ],
        compiler_params=pltpu.CompilerParams(use_tc_tiling_on_sc=False),
    )
    def body(slots_hbm, idx_hbm, table_hbm, out_hbm, idx_v, rows_v, slots_v, sem):
        wid = lax.axis_index("s") * _NC + lax.axis_index("c")
        base = wid * rows_per_w

        def chunk_body(c, carry):
            off = base + c * _CHUNK
            pltpu.sync_copy(idx_hbm.at[pl.ds(off, _CHUNK)], idx_v)
            gather = pltpu.async_copy(table_hbm.at[idx_v], rows_v, sem)
            pltpu.sync_copy(slots_hbm.at[pl.ds(off, _CHUNK)], slots_v)
            gather.wait()

            def mul_row(r, inner_carry):
                for j in range(_VPR):
                    sl = pl.ds(j * _LANES, _LANES)
                    slots_v[r, sl] = slots_v[r, sl] * rows_v[r, sl]
                return inner_carry

            lax.fori_loop(0, _CHUNK, mul_row, 0)
            pltpu.sync_copy(slots_v, out_hbm.at[pl.ds(off, _CHUNK)])
            return carry

        lax.fori_loop(0, n_chunks, chunk_body, 0)

    return body


def kernel(slots, indices, scale_values):
    b, k, d = slots.shape
    n = b * k
    idx = indices.reshape(n).astype(jnp.int32)
    slots_flat = slots.reshape(n, d)
    out = _build(n)(slots_flat, idx, scale_values)
    return out.reshape(b, k, d)


# speedup vs baseline: 1.0858x; 1.0858x over previous
import functools

import jax
import jax.numpy as jnp
from jax import lax
from jax.experimental import pallas as pl
from jax.experimental.pallas import tpu as pltpu
from jax.experimental.pallas import tpu_sc as plsc

_DIM = 64
_LANES = 16
_NC = 2
_NS = 16
_NW = _NC * _NS
_CHUNK = 128
_VPR = _DIM // _LANES
_NBUF = 2


@functools.lru_cache(maxsize=None)
def _build(n_rows):
    rows_per_w = n_rows // _NW
    n_chunks = rows_per_w // _CHUNK
    mesh = plsc.VectorSubcoreMesh(core_axis_name="c", subcore_axis_name="s",
                                  num_cores=_NC, num_subcores=_NS)

    @functools.partial(
        pl.kernel,
        out_type=jax.ShapeDtypeStruct((n_rows, _DIM), jnp.float32),
        mesh=mesh,
        scratch_types=[
            pltpu.VMEM((n_chunks, _CHUNK), jnp.int32),
            *[pltpu.VMEM((_CHUNK, _DIM), jnp.float32) for _ in range(3 * _NBUF)],
            *[pltpu.SemaphoreType.DMA for _ in range(3 * _NBUF)],
        ],
        compiler_params=pltpu.CompilerParams(use_tc_tiling_on_sc=False),
    )
    def body(slots_hbm, idx_hbm, table_hbm, out_hbm, idx_all,
             rows0, rows1, slots0, slots1, out0, out1,
             gs0, gs1, ss0, ss1, os0, os1):
        rows_v = [rows0, rows1]
        slots_v = [slots0, slots1]
        out_v = [out0, out1]
        gsem = [gs0, gs1]
        ssem = [ss0, ss1]
        osem = [os0, os1]
        wid = lax.axis_index("s") * _NC + lax.axis_index("c")
        base = wid * rows_per_w
        pltpu.sync_copy(idx_hbm.at[wid], idx_all)

        def gather_copy(c, b):
            return pltpu.make_async_copy(
                table_hbm.at[idx_all.at[c]], rows_v[b], gsem[b])

        def slots_copy(c, b):
            return pltpu.make_async_copy(
                slots_hbm.at[pl.ds(base + c * _CHUNK, _CHUNK)], slots_v[b], ssem[b])

        def store_copy(c, b):
            return pltpu.make_async_copy(
                out_v[b], out_hbm.at[pl.ds(base + c * _CHUNK, _CHUNK)], osem[b])

        for b in range(_NBUF):
            gather_copy(b, b).start()
            slots_copy(b, b).start()

        def outer(g, carry):
            for b in range(_NBUF):
                c = g * _NBUF + b
                gather_copy(c, b).wait()
                slots_copy(c, b).wait()

                @pl.when(c >= _NBUF)
                def _():
                    store_copy(c - _NBUF, b).wait()

                @plsc.parallel_loop(0, _CHUNK, unroll=4)
                def _(r):
                    for j in range(_VPR):
                        sl = pl.ds(j * _LANES, _LANES)
                        out_v[b][r, sl] = rows_v[b][r, sl] * slots_v[b][r, sl]

                store_copy(c, b).start()

                @pl.when(c + _NBUF < n_chunks)
                def _():
                    gather_copy(c + _NBUF, b).start()
                    slots_copy(c + _NBUF, b).start()
            return carry

        lax.fori_loop(0, n_chunks // _NBUF, outer, 0)
        for b in range(_NBUF):
            store_copy(n_chunks - _NBUF + b, b).wait()

    return body


def kernel(slots, indices, scale_values):
    b, k, d = slots.shape
    n = b * k
    rows_per_w = n // _NW
    n_chunks = rows_per_w // _CHUNK
    idx = indices.reshape(n).astype(jnp.int32).reshape(_NW, n_chunks, _CHUNK)
    slots_flat = slots.reshape(n, d)
    out = _build(n)(slots_flat, idx, scale_values)
    return out.reshape(b, k, d)
